# baseline (device time: 170775 ns/iter reference)
import jax
import jax.numpy as jnp
from jax import lax
from jax.experimental import pallas as pl
from jax.experimental.pallas import tpu as pltpu

N_DEV = 32
B, SQ, SKV, D_MODEL = 2, 256, 512, 768
H_LOC, DH = 8, 64
ROWS = B * SQ
CHUNK = ROWS // N_DEV


def _allreduce_body(x_ref, out_ref, comm_ref, rs_send, rs_recv, ag_send, ag_recv):
    my = lax.axis_index("i")
    left = lax.rem(my + N_DEV - 1, N_DEV)
    right = lax.rem(my + 1, N_DEV)

    barrier = pltpu.get_barrier_semaphore()
    for nbr in (left, right):
        pl.semaphore_signal(
            barrier, inc=1, device_id=(nbr,),
            device_id_type=pl.DeviceIdType.MESH,
        )
    pl.semaphore_wait(barrier, 2)

    out_ref[...] = x_ref[...]

    for s in range(N_DEV - 1):
        send_c = lax.rem(my - s + N_DEV, N_DEV)
        recv_c = lax.rem(my - s - 1 + N_DEV, N_DEV)
        rdma = pltpu.make_async_remote_copy(
            src_ref=out_ref.at[send_c],
            dst_ref=comm_ref.at[s],
            send_sem=rs_send.at[s],
            recv_sem=rs_recv.at[s],
            device_id=(right,),
            device_id_type=pl.DeviceIdType.MESH,
        )
        rdma.start()
        rdma.wait()
        out_ref[recv_c] = out_ref[recv_c] + comm_ref[s]

    for s in range(N_DEV - 1):
        send_c = lax.rem(my + 1 - s + 2 * N_DEV, N_DEV)
        rdma = pltpu.make_async_remote_copy(
            src_ref=out_ref.at[send_c],
            dst_ref=out_ref.at[send_c],
            send_sem=ag_send.at[s],
            recv_sem=ag_recv.at[s],
            device_id=(right,),
            device_id_type=pl.DeviceIdType.MESH,
        )
        rdma.start()
        rdma.wait()


def _ring_allreduce(partial):
    out = pl.pallas_call(
        _allreduce_body,
        out_shape=jax.ShapeDtypeStruct((N_DEV, CHUNK, D_MODEL), jnp.float32),
        in_specs=[pl.BlockSpec(memory_space=pltpu.VMEM)],
        out_specs=pl.BlockSpec(memory_space=pltpu.VMEM),
        scratch_shapes=[
            pltpu.VMEM((N_DEV - 1, CHUNK, D_MODEL), jnp.float32),
            pltpu.SemaphoreType.DMA((N_DEV - 1,)),
            pltpu.SemaphoreType.DMA((N_DEV - 1,)),
            pltpu.SemaphoreType.DMA((N_DEV - 1,)),
            pltpu.SemaphoreType.DMA((N_DEV - 1,)),
        ],
        compiler_params=pltpu.CompilerParams(collective_id=0),
    )(partial.reshape(N_DEV, CHUNK, D_MODEL))
    return out.reshape(ROWS, D_MODEL)


def kernel(x, Wq, Wo, K_ext, V_ext):
    xb = x.reshape(ROWS, D_MODEL).astype(jnp.bfloat16)
    q = (xb @ Wq.astype(jnp.bfloat16)).reshape(B, SQ, H_LOC, DH)
    k = K_ext.astype(jnp.bfloat16)
    v = V_ext.astype(jnp.bfloat16)
    s = jnp.einsum(
        "bihd,bjhd->bhij", q, k, preferred_element_type=jnp.float32
    ) * 0.125
    p = jax.nn.softmax(s, axis=-1).astype(jnp.bfloat16)
    o = jnp.einsum(
        "bhij,bjhd->bihd", p, v, preferred_element_type=jnp.float32
    )
    o = o.reshape(ROWS, H_LOC * DH).astype(jnp.bfloat16)
    partial = jnp.dot(
        o, Wo.astype(jnp.bfloat16), preferred_element_type=jnp.float32
    )
    return _ring_allreduce(partial).reshape(B, SQ, D_MODEL)


# device time: 48929 ns/iter; 3.4903x vs baseline; 3.4903x over previous
import jax
import jax.numpy as jnp
from jax import lax
from jax.experimental import pallas as pl
from jax.experimental.pallas import tpu as pltpu

N_DEV = 32
B, SQ, SKV, D_MODEL = 2, 256, 512, 768
H_LOC, DH = 8, 64
ROWS = B * SQ
CHUNK = ROWS // N_DEV


def _allreduce_body(
    x_ref, out_ref, conv_ref, rs_buf, ag_buf,
    rs_send, rs_recv, ag_send, ag_recv,
):
    my = lax.axis_index("i")

    barrier = pltpu.get_barrier_semaphore()
    for d in range(1, N_DEV):
        pl.semaphore_signal(
            barrier, inc=1,
            device_id=(lax.rem(my + d, N_DEV),),
            device_id_type=pl.DeviceIdType.MESH,
        )
    pl.semaphore_wait(barrier, N_DEV - 1)

    conv_ref[...] = x_ref[...].astype(jnp.bfloat16)

    rs = []
    for d in range(1, N_DEV):
        tgt = lax.rem(my + d, N_DEV)
        rdma = pltpu.make_async_remote_copy(
            src_ref=conv_ref.at[tgt],
            dst_ref=rs_buf.at[N_DEV - d],
            send_sem=rs_send.at[d],
            recv_sem=rs_recv.at[N_DEV - d],
            device_id=(tgt,),
            device_id_type=pl.DeviceIdType.MESH,
        )
        rdma.start()
        rs.append(rdma)
    for rdma in rs:
        rdma.wait_recv()
    for rdma in rs:
        rdma.wait_send()

    acc = x_ref[my]
    for j in range(1, N_DEV):
        acc = acc + rs_buf[j].astype(jnp.float32)
    ag_buf[my] = acc.astype(jnp.bfloat16)

    ag = []
    for d in range(1, N_DEV):
        tgt = lax.rem(my + d, N_DEV)
        rdma = pltpu.make_async_remote_copy(
            src_ref=ag_buf.at[my],
            dst_ref=ag_buf.at[my],
            send_sem=ag_send.at[d],
            recv_sem=ag_recv.at[N_DEV - d],
            device_id=(tgt,),
            device_id_type=pl.DeviceIdType.MESH,
        )
        rdma.start()
        ag.append(rdma)
    for rdma in ag:
        rdma.wait_recv()
    for rdma in ag:
        rdma.wait_send()

    out_ref[...] = ag_buf[...].astype(jnp.float32)


def _allreduce(partial):
    out = pl.pallas_call(
        _allreduce_body,
        out_shape=jax.ShapeDtypeStruct((N_DEV, CHUNK, D_MODEL), jnp.float32),
        in_specs=[pl.BlockSpec(memory_space=pltpu.VMEM)],
        out_specs=pl.BlockSpec(memory_space=pltpu.VMEM),
        scratch_shapes=[
            pltpu.VMEM((N_DEV, CHUNK, D_MODEL), jnp.bfloat16),
            pltpu.VMEM((N_DEV, CHUNK, D_MODEL), jnp.bfloat16),
            pltpu.VMEM((N_DEV, CHUNK, D_MODEL), jnp.bfloat16),
            pltpu.SemaphoreType.DMA((N_DEV,)),
            pltpu.SemaphoreType.DMA((N_DEV,)),
            pltpu.SemaphoreType.DMA((N_DEV,)),
            pltpu.SemaphoreType.DMA((N_DEV,)),
        ],
        compiler_params=pltpu.CompilerParams(collective_id=0),
    )(partial.reshape(N_DEV, CHUNK, D_MODEL))
    return out.reshape(ROWS, D_MODEL)


def kernel(x, Wq, Wo, K_ext, V_ext):
    xb = x.reshape(ROWS, D_MODEL).astype(jnp.bfloat16)
    q = (xb @ Wq.astype(jnp.bfloat16)).reshape(B, SQ, H_LOC, DH)
    k = K_ext.astype(jnp.bfloat16)
    v = V_ext.astype(jnp.bfloat16)
    s = jnp.einsum(
        "bihd,bjhd->bhij", q, k, preferred_element_type=jnp.float32
    ) * 0.125
    p = jax.nn.softmax(s, axis=-1).astype(jnp.bfloat16)
    o = jnp.einsum(
        "bhij,bjhd->bihd", p, v, preferred_element_type=jnp.float32
    )
    o = o.reshape(ROWS, H_LOC * DH).astype(jnp.bfloat16)
    partial = jnp.dot(
        o, Wo.astype(jnp.bfloat16), preferred_element_type=jnp.float32
    )
    return _allreduce(partial).reshape(B, SQ, D_MODEL)
